# Initial kernel scaffold; baseline (speedup 1.0000x reference)
import jax
import jax.numpy as jnp
from jax.experimental import pallas as pl


def kernel(filler_list, role_list, filler_emb, role_emb, W, b):
    raise NotImplementedError("write your pallas kernel here")


# speedup vs baseline: 8.2065x; 8.2065x over previous
import functools

import jax
import jax.numpy as jnp
from jax import lax
from jax.experimental import pallas as pl
from jax.experimental.pallas import tpu as pltpu
from jax.experimental.pallas import tpu_sc as plsc

B, S = 4096, 50
N = B * S
FD, RD, OUT = 32, 32, 128
NR = 50

NC, NS = 2, 16
NW = NC * NS
PER_W = N // NW
CH = 800
NCHUNK = PER_W // CH

NB_BLK = 128
ROWS_BLK = NB_BLK * S
GRID = B // NB_BLK


def _sc_gather(table, idx):
    mesh = plsc.VectorSubcoreMesh(core_axis_name="c", subcore_axis_name="s")

    @functools.partial(
        pl.kernel,
        out_type=jax.ShapeDtypeStruct((N, FD), jnp.float32),
        mesh=mesh,
        scratch_types=[
            pltpu.VMEM((CH,), jnp.int32),
            pltpu.VMEM((CH, FD), jnp.float32),
            pltpu.SemaphoreType.DMA,
        ],
        compiler_params=pltpu.CompilerParams(use_tc_tiling_on_sc=False),
    )
    def k(table_hbm, idx_hbm, out_hbm, idx_v, rows_v, sem):
        wid = lax.axis_index("s") * NC + lax.axis_index("c")
        base = wid * PER_W

        @pl.loop(0, NCHUNK)
        def _(ci):
            off = base + ci * CH
            pltpu.sync_copy(idx_hbm.at[pl.ds(off, CH)], idx_v)
            pltpu.async_copy(table_hbm.at[idx_v], rows_v, sem).wait()
            pltpu.sync_copy(rows_v, out_hbm.at[pl.ds(off, CH)])

    return k(table, idx)


def _tc_body(f_ref, r_ref, remb_ref, w_ref, b_ref, o_ref):
    F2 = f_ref[...]
    rid = r_ref[...]
    kio = lax.broadcasted_iota(jnp.int32, (ROWS_BLK, NR), 1)
    oh = (rid == kio).astype(jnp.float32)
    R2 = jnp.dot(oh, remb_ref[...], preferred_element_type=jnp.float32)
    F3 = F2.reshape(NB_BLK, S, FD)
    R3 = R2.reshape(NB_BLK, S, RD)
    bound = lax.dot_general(
        F3, R3, (((1,), (1,)), ((0,), (0,))),
        preferred_element_type=jnp.float32)
    flat = bound.reshape(NB_BLK, FD * RD)
    o_ref[...] = jnp.dot(flat, w_ref[...],
                         preferred_element_type=jnp.float32) + b_ref[...]


def _tc_compute(gathered, role_flat, role_emb, w_t, b2):
    return pl.pallas_call(
        _tc_body,
        grid=(GRID,),
        in_specs=[
            pl.BlockSpec((ROWS_BLK, FD), lambda i: (i, 0)),
            pl.BlockSpec((ROWS_BLK, 1), lambda i: (i, 0)),
            pl.BlockSpec((NR, RD), lambda i: (0, 0)),
            pl.BlockSpec((FD * RD, OUT), lambda i: (0, 0)),
            pl.BlockSpec((1, OUT), lambda i: (0, 0)),
        ],
        out_specs=pl.BlockSpec((NB_BLK, OUT), lambda i: (i, 0)),
        out_shape=jax.ShapeDtypeStruct((B, OUT), jnp.float32),
    )(gathered, role_flat, role_emb, w_t, b2)


@jax.jit
def kernel(filler_list, role_list, filler_emb, role_emb, W, b):
    idx = filler_list.reshape(-1)
    gathered = _sc_gather(filler_emb, idx)
    return _tc_compute(gathered, role_list.reshape(-1, 1), role_emb,
                       W.T, b.reshape(1, -1))
